# baseline (speedup 1.0000x reference)
import jax
import jax.numpy as jnp
from jax import lax
from jax.experimental import pallas as pl
from jax.experimental.pallas import tpu as pltpu
from jax.experimental.pallas import tpu_sc as plsc

B, L, D = 16, 4096, 1024
MAXC = 2048
NC, NS = 2, 16
NW = NC * NS
DSUB = D // NW
TT = 256
NTILES = L // TT


def _body(x_hbm, b_hbm, out_hbm, cnt_hbm, bbuf, xbuf0, xbuf1, obuf, pbuf,
          posA, cbuf, cibuf, semb, sem0, sem1, semo):
    cid = lax.axis_index("c")
    sid = lax.axis_index("s")
    wid = sid * NC + cid
    ds0 = pl.multiple_of(wid * DSUB, DSUB)
    lanes = lax.iota(jnp.int32, 16)
    bufs = (xbuf0, xbuf1)
    sems = (sem0, sem1)
    zv = jnp.zeros((16,), jnp.float32)

    def out_dst(b):
        return out_hbm.at[b, :, pl.ds(ds0, DSUB)]

    pbuf[0, pl.ds(0, 16)] = zv
    pbuf[0, pl.ds(16, 16)] = zv

    def batch_body(b, bcarry):
        def xsrc(ti):
            return x_hbm.at[b, pl.ds(ti * TT, TT), pl.ds(ds0, DSUB)]

        pltpu.async_copy(b_hbm.at[b], bbuf, semb)
        pltpu.async_copy(xsrc(0), xbuf0, sem0)
        pltpu.async_copy(xsrc(1), xbuf1, sem1)

        lv = jnp.full((16,), L, jnp.int32)

        def fill_a(i, cc):
            posA[pl.ds(i * 16, 16)] = lv
            return cc

        lax.fori_loop(0, (MAXC + 32) // 16, fill_a, 0, unroll=4)

        pltpu.make_async_copy(b_hbm.at[b], bbuf, semb).wait()

        def comp(g, ptr):
            bv = bbuf[pl.ds(g * 16, 16)]
            msk = bv > 0.5
            posv = lanes + g * 16
            cs = plsc.cumsum(msk.astype(jnp.int32)) + ptr
            plsc.store_scatter(posA, [cs - 1], posv, mask=msk)
            cbuf[pl.ds(g * 16, 16)] = cs
            return cs[15]

        nb = lax.fori_loop(0, L // 16, comp, jnp.int32(0), unroll=2)

        @pl.when(b > 0)
        def _():
            pltpu.make_async_copy(obuf.at[pl.ds(1, MAXC)], out_dst(b),
                                  semo).wait()

        def prefix_grp(xbuf):
            def grp(g, gc):
                p_a, p_b = gc
                t0 = g * 16
                for i in range(16):
                    p_a = p_a + xbuf[t0 + i, pl.ds(0, 16)]
                    p_b = p_b + xbuf[t0 + i, pl.ds(16, 16)]
                    pbuf[t0 + i + 1, pl.ds(0, 16)] = p_a
                    pbuf[t0 + i + 1, pl.ds(16, 16)] = p_b
                return p_a, p_b

            return grp

        def seg_emit(t0, c_a, c_b, s_hi):
            def seg(s, sc):
                pe_a, pe_b = sc
                pv = posA[pl.ds(s + 1, 16)]
                el = pv[0] - t0
                ve_a = pbuf[el, pl.ds(0, 16)] + c_a
                ve_b = pbuf[el, pl.ds(16, 16)] + c_b
                obuf[s + 1, pl.ds(0, 16)] = ve_a - pe_a
                obuf[s + 1, pl.ds(16, 16)] = ve_b - pe_b
                return ve_a, ve_b

            return seg

        carry = (zv, zv, zv, zv, jnp.int32(-1))
        for ti in range(NTILES):
            buf = bufs[ti % 2]
            sem = sems[ti % 2]
            pltpu.make_async_copy(xsrc(ti), buf, sem).wait()
            c_a, c_b, pe_a, pe_b, s_lo = carry
            p_a, p_b = lax.fori_loop(0, TT // 16, prefix_grp(buf), (zv, zv))
            if ti + 2 < NTILES:
                pltpu.async_copy(xsrc(ti + 2), buf, sem)
            if ti + 1 < NTILES:
                cnext = cbuf[pl.ds((ti + 1) * TT, 16)]
                s_hi = jnp.minimum(cnext[0] - 1, MAXC + 1)
            else:
                s_hi = jnp.minimum(nb, MAXC + 1)
            t0 = ti * TT
            seg = seg_emit(t0, c_a, c_b, s_hi)
            nfull = jnp.maximum(s_hi - s_lo, 0) // 8

            def seg_block(k, sc):
                pe_a, pe_b = sc
                s = s_lo + k * 8
                pv = posA[pl.ds(s + 1, 16)]
                va = []
                vb = []
                for j in range(8):
                    el = pv[j] - t0
                    va.append(pbuf[el, pl.ds(0, 16)] + c_a)
                    vb.append(pbuf[el, pl.ds(16, 16)] + c_b)
                for j in range(8):
                    obuf[s + j + 1, pl.ds(0, 16)] = va[j] - pe_a
                    obuf[s + j + 1, pl.ds(16, 16)] = vb[j] - pe_b
                    pe_a = va[j]
                    pe_b = vb[j]
                return pe_a, pe_b

            pe_a, pe_b = lax.fori_loop(0, nfull, seg_block, (pe_a, pe_b))
            pe_a, pe_b = lax.fori_loop(
                s_lo + nfull * 8, s_hi, seg, (pe_a, pe_b))
            carry = (c_a + p_a, c_b + p_b, pe_a, pe_b,
                     jnp.maximum(s_lo, s_hi))

        def div_grp(g, cc):
            r0 = g * 16
            pa = posA[pl.ds(r0, 16)]
            pb = posA[pl.ds(r0 + 1, 16)]
            cv = pb - pa
            cibuf[pl.ds(r0, 16)] = cv
            cvf = cv.astype(jnp.float32)
            fac = jnp.where(cv > 0, 1.0 / jnp.maximum(cvf, 1.0), 0.0)
            for i in range(16):
                den = jnp.full((16,), fac[i], jnp.float32)
                obuf[r0 + 1 + i, pl.ds(0, 16)] = (
                    obuf[r0 + 1 + i, pl.ds(0, 16)] * den)
                obuf[r0 + 1 + i, pl.ds(16, 16)] = (
                    obuf[r0 + 1 + i, pl.ds(16, 16)] * den)
            return cc

        lax.fori_loop(0, MAXC // 16, div_grp, 0)

        pltpu.async_copy(obuf.at[pl.ds(1, MAXC)], out_dst(b), semo)

        @pl.when(wid == 0)
        def _():
            pltpu.sync_copy(cibuf, cnt_hbm.at[b])

        return bcarry

    lax.fori_loop(0, B, batch_body, 0)
    pltpu.make_async_copy(obuf.at[pl.ds(1, MAXC)], out_dst(B - 1),
                          semo).wait()


@jax.jit
def kernel(x, boundaries):
    mesh = plsc.VectorSubcoreMesh(core_axis_name="c", subcore_axis_name="s")
    f = pl.kernel(
        _body,
        out_type=(
            jax.ShapeDtypeStruct((B, MAXC, D), jnp.float32),
            jax.ShapeDtypeStruct((B, MAXC), jnp.int32),
        ),
        mesh=mesh,
        compiler_params=pltpu.CompilerParams(
            use_tc_tiling_on_sc=False, needs_layout_passes=False),
        scratch_types=[
            pltpu.VMEM((L,), jnp.float32),
            pltpu.VMEM((TT, DSUB), jnp.float32),
            pltpu.VMEM((TT, DSUB), jnp.float32),
            pltpu.VMEM((MAXC + 2, DSUB), jnp.float32),
            pltpu.VMEM((TT + 1, DSUB), jnp.float32),
            pltpu.VMEM((L + 16,), jnp.int32),
            pltpu.VMEM((L,), jnp.int32),
            pltpu.VMEM((MAXC,), jnp.int32),
            pltpu.SemaphoreType.DMA,
            pltpu.SemaphoreType.DMA,
            pltpu.SemaphoreType.DMA,
            pltpu.SemaphoreType.DMA,
        ],
    )
    return f(x, boundaries)


# speedup vs baseline: 1.3836x; 1.3836x over previous
import jax
import jax.numpy as jnp
from jax import lax
from jax.experimental import pallas as pl
from jax.experimental.pallas import tpu as pltpu
from jax.experimental.pallas import tpu_sc as plsc

B, L, D = 16, 4096, 1024
MAXC = 2048
NC, NS = 2, 16
NW = NC * NS
DSUB = D // NW
TT = 256
NTILES = L // TT


def _body(x_hbm, b_hbm, out_hbm, cnt_hbm, bbuf, xbuf0, xbuf1, obuf, pbuf,
          posA, cbuf, cibuf, semb, sem0, sem1, semo):
    cid = lax.axis_index("c")
    sid = lax.axis_index("s")
    wid = sid * NC + cid
    ds0 = pl.multiple_of(wid * DSUB, DSUB)
    lanes = lax.iota(jnp.int32, 16)
    bufs = (xbuf0, xbuf1)
    sems = (sem0, sem1)
    zv = jnp.zeros((16,), jnp.float32)

    def out_dst(b):
        return out_hbm.at[b, :, pl.ds(ds0, DSUB)]

    pbuf[0, pl.ds(0, 16)] = zv
    pbuf[0, pl.ds(16, 16)] = zv

    def batch_body(b, bcarry):
        def xsrc(ti):
            return x_hbm.at[b, pl.ds(ti * TT, TT), pl.ds(ds0, DSUB)]

        pltpu.async_copy(b_hbm.at[b], bbuf, semb)
        pltpu.async_copy(xsrc(0), xbuf0, sem0)
        pltpu.async_copy(xsrc(1), xbuf1, sem1)

        lv = jnp.full((16,), L, jnp.int32)

        def fill_a(i, cc):
            posA[pl.ds(i * 16, 16)] = lv
            return cc

        lax.fori_loop(0, (MAXC + 32) // 16, fill_a, 0, unroll=4)

        pltpu.make_async_copy(b_hbm.at[b], bbuf, semb).wait()

        def comp(g, ptr):
            bv = bbuf[pl.ds(g * 16, 16)]
            msk = bv > 0.5
            posv = lanes + g * 16
            cs = plsc.cumsum(msk.astype(jnp.int32)) + ptr
            plsc.store_scatter(posA, [cs], posv, mask=msk)
            cbuf[pl.ds(g * 16, 16)] = cs
            return cs[15]

        nb = lax.fori_loop(0, L // 16, comp, jnp.int32(0), unroll=2)

        @pl.when(b > 0)
        def _():
            pltpu.make_async_copy(obuf.at[pl.ds(1, MAXC)], out_dst(b),
                                  semo).wait()

        def prefix_grp(xbuf):
            def grp(g, gc):
                p_a, p_b = gc
                t0 = g * 16
                for i in range(16):
                    p_a = p_a + xbuf[t0 + i, pl.ds(0, 16)]
                    p_b = p_b + xbuf[t0 + i, pl.ds(16, 16)]
                    pbuf[t0 + i + 1, pl.ds(0, 16)] = p_a
                    pbuf[t0 + i + 1, pl.ds(16, 16)] = p_b
                return p_a, p_b

            return grp

        def seg_emit(t0, c_a, c_b, s_hi):
            def seg(s, sc):
                pe_a, pe_b = sc
                pv = posA[pl.ds(s + 1, 16)]
                el = pv[1] - t0
                cnt = pv[1] - pv[0]
                cntf = jnp.maximum(cnt, 1).astype(jnp.float32)
                denv = jnp.full((16,), cntf, jnp.float32)
                facv = jnp.where(cnt > 0, 1.0 / denv, 0.0)
                ve_a = pbuf[el, pl.ds(0, 16)] + c_a
                ve_b = pbuf[el, pl.ds(16, 16)] + c_b
                obuf[s + 1, pl.ds(0, 16)] = (ve_a - pe_a) * facv
                obuf[s + 1, pl.ds(16, 16)] = (ve_b - pe_b) * facv
                return ve_a, ve_b

            return seg

        carry = (zv, zv, zv, zv, jnp.int32(-1))
        for ti in range(NTILES):
            buf = bufs[ti % 2]
            sem = sems[ti % 2]
            pltpu.make_async_copy(xsrc(ti), buf, sem).wait()
            c_a, c_b, pe_a, pe_b, s_lo = carry
            p_a, p_b = lax.fori_loop(0, TT // 16, prefix_grp(buf), (zv, zv))
            if ti + 2 < NTILES:
                pltpu.async_copy(xsrc(ti + 2), buf, sem)
            if ti + 1 < NTILES:
                cnext = cbuf[pl.ds((ti + 1) * TT, 16)]
                s_hi = jnp.minimum(cnext[0] - 1, MAXC + 1)
            else:
                s_hi = jnp.minimum(nb, MAXC + 1)
            t0 = ti * TT
            seg = seg_emit(t0, c_a, c_b, s_hi)
            nfull = jnp.maximum(s_hi - s_lo, 0) // 8

            def seg_block(k, sc):
                pe_a, pe_b = sc
                s = s_lo + k * 8
                pv = posA[pl.ds(s + 1, 16)]
                pv2 = posA[pl.ds(s + 2, 16)]
                cvv = pv2 - pv
                cvf = cvv.astype(jnp.float32)
                facv = jnp.where(cvv > 0, 1.0 / jnp.maximum(cvf, 1.0), 0.0)
                va = []
                vb = []
                for j in range(8):
                    el = pv[j + 1] - t0
                    va.append(pbuf[el, pl.ds(0, 16)] + c_a)
                    vb.append(pbuf[el, pl.ds(16, 16)] + c_b)
                for j in range(8):
                    den = jnp.full((16,), facv[j], jnp.float32)
                    obuf[s + j + 1, pl.ds(0, 16)] = (va[j] - pe_a) * den
                    obuf[s + j + 1, pl.ds(16, 16)] = (vb[j] - pe_b) * den
                    pe_a = va[j]
                    pe_b = vb[j]
                return pe_a, pe_b

            pe_a, pe_b = lax.fori_loop(0, nfull, seg_block, (pe_a, pe_b))
            pe_a, pe_b = lax.fori_loop(
                s_lo + nfull * 8, s_hi, seg, (pe_a, pe_b))
            carry = (c_a + p_a, c_b + p_b, pe_a, pe_b,
                     jnp.maximum(s_lo, s_hi))

        def cnt_grp(g, cc):
            r0 = g * 16
            cibuf[pl.ds(r0, 16)] = (posA[pl.ds(r0 + 2, 16)]
                                    - posA[pl.ds(r0 + 1, 16)])
            return cc

        lax.fori_loop(0, MAXC // 16, cnt_grp, 0, unroll=4)

        def zfill(r, cc):
            obuf[r, pl.ds(0, 16)] = zv
            obuf[r, pl.ds(16, 16)] = zv
            return cc

        lax.fori_loop(jnp.minimum(nb, MAXC + 1) + 1, MAXC + 2, zfill, 0)

        pltpu.async_copy(obuf.at[pl.ds(1, MAXC)], out_dst(b), semo)

        @pl.when(wid == 0)
        def _():
            pltpu.sync_copy(cibuf, cnt_hbm.at[b])

        return bcarry

    lax.fori_loop(0, B, batch_body, 0)
    pltpu.make_async_copy(obuf.at[pl.ds(1, MAXC)], out_dst(B - 1),
                          semo).wait()


@jax.jit
def kernel(x, boundaries):
    mesh = plsc.VectorSubcoreMesh(core_axis_name="c", subcore_axis_name="s")
    f = pl.kernel(
        _body,
        out_type=(
            jax.ShapeDtypeStruct((B, MAXC, D), jnp.float32),
            jax.ShapeDtypeStruct((B, MAXC), jnp.int32),
        ),
        mesh=mesh,
        compiler_params=pltpu.CompilerParams(
            use_tc_tiling_on_sc=False, needs_layout_passes=False),
        scratch_types=[
            pltpu.VMEM((L,), jnp.float32),
            pltpu.VMEM((TT, DSUB), jnp.float32),
            pltpu.VMEM((TT, DSUB), jnp.float32),
            pltpu.VMEM((MAXC + 2, DSUB), jnp.float32),
            pltpu.VMEM((TT + 1, DSUB), jnp.float32),
            pltpu.VMEM((L + 16,), jnp.int32),
            pltpu.VMEM((L,), jnp.int32),
            pltpu.VMEM((MAXC,), jnp.int32),
            pltpu.SemaphoreType.DMA,
            pltpu.SemaphoreType.DMA,
            pltpu.SemaphoreType.DMA,
            pltpu.SemaphoreType.DMA,
        ],
    )
    return f(x, boundaries)
